# baseline (speedup 1.0000x reference)
import jax
import jax.numpy as jnp
from jax import lax
from jax.experimental import pallas as pl
from jax.experimental.pallas import tpu as pltpu

NB = 256
H = 224
ROWS = H * H
BS = 3584


def _body(t_sm, gam_sm, s1_sm, y_ref, n_ref, oy_ref, on_ref,
          coef_vmem, coef_smem, sem):
    @pl.when(pl.program_id(0) == 0)
    def _():
        def lp(b, carry):
            idx = t_sm[b]
            coef_smem[0, b] = gam_sm[idx]
            coef_smem[1, b] = s1_sm[idx]
            return carry

        lax.fori_loop(0, NB, lp, 0)
        cp = pltpu.make_async_copy(coef_smem, coef_vmem, sem)
        cp.start()
        cp.wait()

    g = coef_vmem[0:1, :]
    s = coef_vmem[1:2, :]
    nv = n_ref[...]
    oy_ref[...] = g * y_ref[...] + s * nv
    on_ref[...] = nv


_scale_add_call = pl.pallas_call(
    _body,
    grid_spec=pltpu.PrefetchScalarGridSpec(
        num_scalar_prefetch=3,
        grid=(ROWS // BS,),
        in_specs=[
            pl.BlockSpec((BS, NB), lambda i, t, g, s: (i, 0)),
            pl.BlockSpec((BS, NB), lambda i, t, g, s: (i, 0)),
        ],
        out_specs=[
            pl.BlockSpec((BS, NB), lambda i, t, g, s: (i, 0)),
            pl.BlockSpec((BS, NB), lambda i, t, g, s: (i, 0)),
        ],
        scratch_shapes=[
            pltpu.VMEM((2, NB), jnp.float32),
            pltpu.SMEM((2, NB), jnp.float32),
            pltpu.SemaphoreType.DMA,
        ],
    ),
    out_shape=[
        jax.ShapeDtypeStruct((ROWS, NB), jnp.float32),
        jax.ShapeDtypeStruct((ROWS, NB), jnp.float32),
    ],
)


def kernel(y, noise, t, gammas, sqrt_one_minus_gammas, sqrt_gammas):
    t32 = t.astype(jnp.int32)
    y2 = y.transpose(1, 2, 3, 0).reshape(ROWS, NB)
    n2 = noise.transpose(1, 2, 3, 0).reshape(ROWS, NB)
    oy2, on2 = _scale_add_call(t32, gammas, sqrt_one_minus_gammas, y2, n2)
    oy = oy2.reshape(1, H, H, NB).transpose(3, 0, 1, 2)
    on = on2.reshape(1, H, H, NB).transpose(3, 0, 1, 2)
    return oy, on
